# baseline (speedup 1.0000x reference)
import functools

import jax
import jax.numpy as jnp
from jax import lax
from jax.experimental import pallas as pl
from jax.experimental.pallas import tpu as pltpu
from jax.experimental.pallas import tpu_sc as plsc

N = 10000
D = 128
E = 320000

NC = 2
NS = 16
L = 16

N_PAD = 10240
ROWS_PER_TILE = N_PAD // NS
CH = 128
GS = 8
NG = 10
CHUNKS_PER_TILE = GS * NG
E_TILE = CH * CHUNKS_PER_TILE
E_PAD = E_TILE * NC * NS
N_BLK = 1024
G_BLK = N_PAD // N_BLK

_mesh = plsc.VectorSubcoreMesh(core_axis_name="c", subcore_axis_name="s")


@functools.partial(
    pl.kernel,
    out_type=jax.ShapeDtypeStruct((NC, N_PAD), jnp.float32),
    mesh=_mesh,
    scratch_types=[
        pltpu.VMEM((NG, GS, 2, CH), jnp.int32),
        pltpu.VMEM((ROWS_PER_TILE,), jnp.float32),
        pltpu.VMEM_SHARED((N_PAD,), jnp.float32),
        pltpu.SemaphoreType.DMA,
    ],
)
def _deg_call(rc_hbm, out_hbm, idx_v, stage_v, deg_sh, sem):
    c = lax.axis_index("c")
    s = lax.axis_index("s")
    wid = c * NS + s

    idx_cp = pltpu.async_copy(rc_hbm.at[wid], idx_v, sem)

    def _zero(i, _):
        stage_v[pl.ds(i * L, L)] = jnp.zeros((L,), jnp.float32)
        return 0
    lax.fori_loop(0, ROWS_PER_TILE // L, _zero, 0)
    pltpu.sync_copy(stage_v, deg_sh.at[pl.ds(s * ROWS_PER_TILE, ROWS_PER_TILE)])

    def _ones(i, _):
        stage_v[pl.ds(i * L, L)] = jnp.full((L,), 1.0, jnp.float32)
        return 0
    lax.fori_loop(0, CH // L, _ones, 0)

    idx_cp.wait()
    plsc.subcore_barrier()

    def _step(j, _):
        g = jax.lax.div(j, GS)
        k = jax.lax.rem(j, GS)
        pltpu.async_copy(stage_v.at[pl.ds(0, CH)], deg_sh.at[idx_v.at[g, k, 1]],
                         sem, add=True)
        return 0
    lax.fori_loop(0, CHUNKS_PER_TILE, _step, 0)

    def _drain(j, _):
        pltpu.make_async_copy(stage_v.at[pl.ds(0, CH)],
                              deg_sh.at[idx_v.at[0, 0, 1]], sem).wait()
        return 0
    lax.fori_loop(0, CHUNKS_PER_TILE, _drain, 0)

    plsc.subcore_barrier()
    pltpu.sync_copy(
        deg_sh.at[pl.ds(s * ROWS_PER_TILE, ROWS_PER_TILE)],
        out_hbm.at[c, pl.ds(s * ROWS_PER_TILE, ROWS_PER_TILE)],
    )


@functools.partial(
    pl.kernel,
    out_type=jax.ShapeDtypeStruct((NC, N_PAD, D), jnp.float32),
    mesh=_mesh,
    scratch_types=[
        pltpu.VMEM((2, GS, 2, CH), jnp.int32),
        pltpu.VMEM((2, CH, D), jnp.float32),
        pltpu.VMEM_SHARED((N_PAD, D), jnp.float32),
        pltpu.SemaphoreType.DMA,
        pltpu.SemaphoreType.DMA,
        pltpu.SemaphoreType.DMA,
    ],
)
def _edge_call(g_hbm, rc_hbm, out_hbm, idx_v, rows_v, acc_sh, gsem, ssem, isem):
    c = lax.axis_index("c")
    s = lax.axis_index("s")
    wid = c * NS + s
    last = CHUNKS_PER_TILE - 1

    pltpu.sync_copy(rc_hbm.at[wid, 0], idx_v.at[0])
    pltpu.async_copy(rc_hbm.at[wid, 1], idx_v.at[1], isem)

    def _zero(i, _):
        rows_v[0, i // (D // L), pl.ds((i % (D // L)) * L, L)] = (
            jnp.zeros((L,), jnp.float32))
        return 0
    lax.fori_loop(0, CH * (D // L), _zero, 0)

    def _zchunk(k, _):
        pltpu.sync_copy(rows_v.at[0], acc_sh.at[pl.ds(s * ROWS_PER_TILE + k * CH, CH)])
        return 0
    lax.fori_loop(0, ROWS_PER_TILE // CH, _zchunk, 0)

    plsc.subcore_barrier()

    pltpu.async_copy(g_hbm.at[idx_v.at[0, 0, 0]], rows_v.at[0], gsem)

    def _step(j, _):
        buf = jax.lax.rem(j, 2)
        grp = jax.lax.div(j, GS)
        k = jax.lax.rem(j, GS)
        slot = jax.lax.rem(grp, 2)
        k1 = jax.lax.rem(j + 1, GS)
        slot1 = jax.lax.rem(jax.lax.div(j + 1, GS), 2)

        pltpu.make_async_copy(g_hbm.at[idx_v.at[slot, k, 0]], rows_v.at[buf],
                              gsem).wait()

        @pl.when(j > 0)
        def _():
            pltpu.make_async_copy(rows_v.at[1 - buf],
                                  acc_sh.at[idx_v.at[slot, k, 1]], ssem).wait()

        @pl.when(jnp.logical_and(k == 0, grp + 1 < NG))
        def _():
            pltpu.async_copy(rc_hbm.at[wid, grp + 1], idx_v.at[1 - slot], isem)

        @pl.when(jnp.logical_and(j < last, k == GS - 1))
        def _():
            pltpu.make_async_copy(rc_hbm.at[wid, 0], idx_v.at[0], isem).wait()

        @pl.when(j < last)
        def _():
            pltpu.async_copy(g_hbm.at[idx_v.at[slot1, k1, 0]], rows_v.at[1 - buf],
                             gsem)

        pltpu.async_copy(rows_v.at[buf], acc_sh.at[idx_v.at[slot, k, 1]], ssem,
                         add=True)
        return 0
    lax.fori_loop(0, CHUNKS_PER_TILE, _step, 0)
    pltpu.make_async_copy(rows_v.at[0], acc_sh.at[idx_v.at[0, 0, 1]], ssem).wait()

    plsc.subcore_barrier()

    def _wchunk(k, _):
        r0 = s * ROWS_PER_TILE + k * CH
        pltpu.sync_copy(acc_sh.at[pl.ds(r0, CH)], out_hbm.at[c, pl.ds(r0, CH)])
        return 0
    lax.fori_loop(0, ROWS_PER_TILE // CH, _wchunk, 0)


def _tc_transform_body(x_ref, w_ref, degp_ref, g_ref):
    deg = degp_ref[0] + degp_ref[1]
    dis = jnp.where(deg > 0, lax.rsqrt(deg), 0.0)
    h = jnp.dot(x_ref[...], w_ref[...], preferred_element_type=jnp.float32)
    g_ref[...] = (h.reshape(N_BLK // D, D, D) * dis[:, :, None]).reshape(N_BLK, D)


def _tc_transform(x_pad, W, degp3):
    return pl.pallas_call(
        _tc_transform_body,
        grid=(G_BLK,),
        in_specs=[
            pl.BlockSpec((N_BLK, D), lambda i: (i, 0)),
            pl.BlockSpec((D, D), lambda i: (0, 0)),
            pl.BlockSpec((NC, N_BLK // D, D), lambda i: (0, i, 0)),
        ],
        out_specs=pl.BlockSpec((N_BLK, D), lambda i: (i, 0)),
        out_shape=jax.ShapeDtypeStruct((N_PAD, D), jnp.float32),
    )(x_pad, W, degp3)


def _tc_final_body(p_ref, degp_ref, b_ref, o_ref):
    deg = degp_ref[0] + degp_ref[1]
    dis = jnp.where(deg > 0, lax.rsqrt(deg), 0.0)
    tot = p_ref[0] + p_ref[1]
    scaled = (tot.reshape(N_BLK // D, D, D) * dis[:, :, None]).reshape(N_BLK, D)
    o_ref[...] = scaled + b_ref[...]


def _tc_final(partials, degp3, b2):
    return pl.pallas_call(
        _tc_final_body,
        grid=(G_BLK,),
        in_specs=[
            pl.BlockSpec((NC, N_BLK, D), lambda i: (0, i, 0)),
            pl.BlockSpec((NC, N_BLK // D, D), lambda i: (0, i, 0)),
            pl.BlockSpec((1, D), lambda i: (0, 0)),
        ],
        out_specs=pl.BlockSpec((N_BLK, D), lambda i: (i, 0)),
        out_shape=jax.ShapeDtypeStruct((N_PAD, D), jnp.float32),
    )(partials, degp3, b2)


def kernel(x, edge_index, W, b):
    ei = edge_index.astype(jnp.int32)
    n_fill = E_PAD - E
    fill = N + (jnp.arange(n_fill, dtype=jnp.int32) % (N_PAD - N))
    row3 = jnp.concatenate([ei[0], fill]).reshape(NC * NS, CHUNKS_PER_TILE, CH)
    col3 = jnp.concatenate([ei[1], fill]).reshape(NC * NS, CHUNKS_PER_TILE, CH)
    rc = jnp.stack([row3, col3], axis=2).reshape(NC * NS, NG, GS, 2, CH)
    x_pad = jnp.pad(x, ((0, N_PAD - N), (0, 0)))

    degp = _deg_call(rc)
    degp3 = degp.reshape(NC, N_PAD // D, D)
    g = _tc_transform(x_pad, W, degp3)
    partials = _edge_call(g, rc)
    out_pad = _tc_final(partials, degp3, b.reshape(1, D))
    return out_pad[:N]


# speedup vs baseline: 1.0017x; 1.0017x over previous
import functools

import jax
import jax.numpy as jnp
from jax import lax
from jax.experimental import pallas as pl
from jax.experimental.pallas import tpu as pltpu
from jax.experimental.pallas import tpu_sc as plsc

N = 10000
D = 128
E = 320000

NC = 2
NS = 16
L = 16

N_PAD = 10240
ROWS_PER_TILE = N_PAD // NS
CH = 128
GS = 8
NG = 10
CHUNKS_PER_TILE = GS * NG
E_TILE = CH * CHUNKS_PER_TILE
E_PAD = E_TILE * NC * NS
N_BLK = 1024
G_BLK = N_PAD // N_BLK

_mesh = plsc.VectorSubcoreMesh(core_axis_name="c", subcore_axis_name="s")


@functools.partial(
    pl.kernel,
    out_type=jax.ShapeDtypeStruct((NC, N_PAD), jnp.float32),
    mesh=_mesh,
    scratch_types=[
        pltpu.VMEM((NG, GS, 2, CH), jnp.int32),
        pltpu.VMEM((ROWS_PER_TILE,), jnp.float32),
        pltpu.VMEM_SHARED((N_PAD,), jnp.float32),
        pltpu.SemaphoreType.DMA,
    ],
)
def _deg_call(rc_hbm, out_hbm, idx_v, stage_v, deg_sh, sem):
    c = lax.axis_index("c")
    s = lax.axis_index("s")
    wid = c * NS + s

    idx_cp = pltpu.async_copy(rc_hbm.at[wid], idx_v, sem)

    def _zero(i, _):
        stage_v[pl.ds(i * L, L)] = jnp.zeros((L,), jnp.float32)
        return 0
    lax.fori_loop(0, ROWS_PER_TILE // L, _zero, 0)
    pltpu.sync_copy(stage_v, deg_sh.at[pl.ds(s * ROWS_PER_TILE, ROWS_PER_TILE)])

    def _ones(i, _):
        stage_v[pl.ds(i * L, L)] = jnp.full((L,), 1.0, jnp.float32)
        return 0
    lax.fori_loop(0, CH // L, _ones, 0)

    idx_cp.wait()
    plsc.subcore_barrier()

    def _step(j, _):
        g = jax.lax.div(j, GS)
        k = jax.lax.rem(j, GS)
        pltpu.async_copy(stage_v.at[pl.ds(0, CH)], deg_sh.at[idx_v.at[g, k, 1]],
                         sem, add=True)
        return 0
    lax.fori_loop(0, CHUNKS_PER_TILE, _step, 0)

    def _drain(j, _):
        pltpu.make_async_copy(stage_v.at[pl.ds(0, CH)],
                              deg_sh.at[idx_v.at[0, 0, 1]], sem).wait()
        return 0
    lax.fori_loop(0, CHUNKS_PER_TILE, _drain, 0)

    plsc.subcore_barrier()
    pltpu.sync_copy(
        deg_sh.at[pl.ds(s * ROWS_PER_TILE, ROWS_PER_TILE)],
        out_hbm.at[c, pl.ds(s * ROWS_PER_TILE, ROWS_PER_TILE)],
    )


@functools.partial(
    pl.kernel,
    out_type=jax.ShapeDtypeStruct((NC, N_PAD, D), jnp.float32),
    mesh=_mesh,
    scratch_types=[
        pltpu.VMEM((2, GS, 2, CH), jnp.int32),
        pltpu.VMEM((2, CH, D), jnp.float32),
        pltpu.VMEM_SHARED((N_PAD, D), jnp.float32),
        pltpu.SemaphoreType.DMA,
        pltpu.SemaphoreType.DMA,
        pltpu.SemaphoreType.DMA,
    ],
)
def _edge_call(g_hbm, rc_hbm, out_hbm, idx_v, rows_v, acc_sh, gsem, ssem, isem):
    c = lax.axis_index("c")
    s = lax.axis_index("s")
    wid = c * NS + s
    last = CHUNKS_PER_TILE - 1

    pltpu.sync_copy(rc_hbm.at[wid, 0], idx_v.at[0])

    def _zero(i, _):
        rows_v[0, i // (D // L), pl.ds((i % (D // L)) * L, L)] = (
            jnp.zeros((L,), jnp.float32))
        return 0
    lax.fori_loop(0, CH * (D // L), _zero, 0)

    def _zchunk(k, _):
        pltpu.sync_copy(rows_v.at[0], acc_sh.at[pl.ds(s * ROWS_PER_TILE + k * CH, CH)])
        return 0
    lax.fori_loop(0, ROWS_PER_TILE // CH, _zchunk, 0)

    plsc.subcore_barrier()

    pltpu.async_copy(g_hbm.at[idx_v.at[0, 0, 0]], rows_v.at[0], gsem)

    def _step(j, _):
        buf = jax.lax.rem(j, 2)
        grp = jax.lax.div(j, GS)
        k = jax.lax.rem(j, GS)
        slot = jax.lax.rem(grp, 2)
        k1 = jax.lax.rem(j + 1, GS)
        slot1 = jax.lax.rem(jax.lax.div(j + 1, GS), 2)

        pltpu.make_async_copy(g_hbm.at[idx_v.at[slot, k, 0]], rows_v.at[buf],
                              gsem).wait()

        @pl.when(j > 0)
        def _():
            pltpu.make_async_copy(rows_v.at[1 - buf],
                                  acc_sh.at[idx_v.at[slot, k, 1]], ssem).wait()

        @pl.when(jnp.logical_and(k == 0, grp + 1 < NG))
        def _():
            pltpu.async_copy(rc_hbm.at[wid, grp + 1], idx_v.at[1 - slot], isem)

        @pl.when(jnp.logical_and(j < last, k == GS - 1))
        def _():
            pltpu.make_async_copy(rc_hbm.at[wid, 0], idx_v.at[0], isem).wait()

        @pl.when(j < last)
        def _():
            pltpu.async_copy(g_hbm.at[idx_v.at[slot1, k1, 0]], rows_v.at[1 - buf],
                             gsem)

        pltpu.async_copy(rows_v.at[buf], acc_sh.at[idx_v.at[slot, k, 1]], ssem,
                         add=True)
        return 0
    lax.fori_loop(0, CHUNKS_PER_TILE, _step, 0)
    pltpu.make_async_copy(rows_v.at[0], acc_sh.at[idx_v.at[0, 0, 1]], ssem).wait()

    plsc.subcore_barrier()

    def _wchunk(k, _):
        r0 = s * ROWS_PER_TILE + k * CH
        pltpu.sync_copy(acc_sh.at[pl.ds(r0, CH)], out_hbm.at[c, pl.ds(r0, CH)])
        return 0
    lax.fori_loop(0, ROWS_PER_TILE // CH, _wchunk, 0)


def _tc_transform_body(x_ref, w_ref, degp_ref, g_ref):
    deg = degp_ref[0] + degp_ref[1]
    dis = jnp.where(deg > 0, lax.rsqrt(deg), 0.0)
    h = jnp.dot(x_ref[...], w_ref[...], preferred_element_type=jnp.float32)
    g_ref[...] = (h.reshape(N_BLK // D, D, D) * dis[:, :, None]).reshape(N_BLK, D)


def _tc_transform(x_pad, W, degp3):
    return pl.pallas_call(
        _tc_transform_body,
        grid=(G_BLK,),
        in_specs=[
            pl.BlockSpec((N_BLK, D), lambda i: (i, 0)),
            pl.BlockSpec((D, D), lambda i: (0, 0)),
            pl.BlockSpec((NC, N_BLK // D, D), lambda i: (0, i, 0)),
        ],
        out_specs=pl.BlockSpec((N_BLK, D), lambda i: (i, 0)),
        out_shape=jax.ShapeDtypeStruct((N_PAD, D), jnp.float32),
    )(x_pad, W, degp3)


def _tc_final_body(p_ref, degp_ref, b_ref, o_ref):
    deg = degp_ref[0] + degp_ref[1]
    dis = jnp.where(deg > 0, lax.rsqrt(deg), 0.0)
    tot = p_ref[0] + p_ref[1]
    scaled = (tot.reshape(N_BLK // D, D, D) * dis[:, :, None]).reshape(N_BLK, D)
    o_ref[...] = scaled + b_ref[...]


def _tc_final(partials, degp3, b2):
    return pl.pallas_call(
        _tc_final_body,
        grid=(G_BLK,),
        in_specs=[
            pl.BlockSpec((NC, N_BLK, D), lambda i: (0, i, 0)),
            pl.BlockSpec((NC, N_BLK // D, D), lambda i: (0, i, 0)),
            pl.BlockSpec((1, D), lambda i: (0, 0)),
        ],
        out_specs=pl.BlockSpec((N_BLK, D), lambda i: (i, 0)),
        out_shape=jax.ShapeDtypeStruct((N_PAD, D), jnp.float32),
    )(partials, degp3, b2)


def kernel(x, edge_index, W, b):
    ei = edge_index.astype(jnp.int32)
    n_fill = E_PAD - E
    fill = N + (jnp.arange(n_fill, dtype=jnp.int32) % (N_PAD - N))
    row3 = jnp.concatenate([ei[0], fill]).reshape(NC * NS, CHUNKS_PER_TILE, CH)
    col3 = jnp.concatenate([ei[1], fill]).reshape(NC * NS, CHUNKS_PER_TILE, CH)
    rc = jnp.stack([row3, col3], axis=2).reshape(NC * NS, NG, GS, 2, CH)
    x_pad = jnp.pad(x, ((0, N_PAD - N), (0, 0)))

    degp = _deg_call(rc)
    degp3 = degp.reshape(NC, N_PAD // D, D)
    g = _tc_transform(x_pad, W, degp3)
    partials = _edge_call(g, rc)
    out_pad = _tc_final(partials, degp3, b.reshape(1, D))
    return out_pad[:N]


# speedup vs baseline: 1.0822x; 1.0804x over previous
import functools

import jax
import jax.numpy as jnp
from jax import lax
from jax.experimental import pallas as pl
from jax.experimental.pallas import tpu as pltpu
from jax.experimental.pallas import tpu_sc as plsc

N = 10000
D = 128
E = 320000

NC = 2
NS = 16
L = 16

N_PAD = 10112
CH = 128
CHUNKS_PER_TILE = 79
E_TILE = CH * CHUNKS_PER_TILE
E_PAD = E_TILE * NC * NS
NRB = 3

STRIPE = 640

_mesh = plsc.VectorSubcoreMesh(core_axis_name="c", subcore_axis_name="s")


@functools.partial(
    pl.kernel,
    out_type=jax.ShapeDtypeStruct((NC, N_PAD), jnp.float32),
    mesh=_mesh,
    scratch_types=[
        pltpu.VMEM((CHUNKS_PER_TILE, CH), jnp.int32),
        pltpu.VMEM((STRIPE,), jnp.float32),
        pltpu.VMEM((CH,), jnp.float32),
        pltpu.VMEM_SHARED((N_PAD,), jnp.float32),
        pltpu.SemaphoreType.DMA,
    ],
)
def _deg_call(col3_hbm, out_hbm, idx_v, zstage_v, ones_v, deg_sh, sem):
    c = lax.axis_index("c")
    s = lax.axis_index("s")
    wid = c * NS + s
    srows = jnp.where(s < NS - 1, STRIPE, N_PAD - (NS - 1) * STRIPE)

    idx_cp = pltpu.async_copy(col3_hbm.at[wid], idx_v, sem)

    def _zero(i, _):
        zstage_v[pl.ds(i * L, L)] = jnp.zeros((L,), jnp.float32)
        return 0
    lax.fori_loop(0, STRIPE // L, _zero, 0)

    @pl.when(s < NS - 1)
    def _():
        pltpu.sync_copy(zstage_v, deg_sh.at[pl.ds(s * STRIPE, STRIPE)])

    @pl.when(s == NS - 1)
    def _():
        pltpu.sync_copy(zstage_v.at[pl.ds(0, N_PAD - (NS - 1) * STRIPE)],
                        deg_sh.at[pl.ds((NS - 1) * STRIPE,
                                        N_PAD - (NS - 1) * STRIPE)])

    def _ones(i, _):
        ones_v[pl.ds(i * L, L)] = jnp.full((L,), 1.0, jnp.float32)
        return 0
    lax.fori_loop(0, CH // L, _ones, 0)

    idx_cp.wait()
    plsc.subcore_barrier()

    def _step(j, _):
        pltpu.async_copy(ones_v, deg_sh.at[idx_v.at[j]], sem, add=True)
        return 0
    lax.fori_loop(0, CHUNKS_PER_TILE, _step, 0)

    def _drain(j, _):
        pltpu.make_async_copy(ones_v, deg_sh.at[idx_v.at[0]], sem).wait()
        return 0
    lax.fori_loop(0, CHUNKS_PER_TILE, _drain, 0)

    plsc.subcore_barrier()

    @pl.when(s < NS - 1)
    def _():
        pltpu.sync_copy(deg_sh.at[pl.ds(s * STRIPE, STRIPE)],
                        out_hbm.at[c, pl.ds(s * STRIPE, STRIPE)])

    @pl.when(s == NS - 1)
    def _():
        pltpu.sync_copy(deg_sh.at[pl.ds((NS - 1) * STRIPE,
                                        N_PAD - (NS - 1) * STRIPE)],
                        out_hbm.at[c, pl.ds((NS - 1) * STRIPE,
                                            N_PAD - (NS - 1) * STRIPE)])


@functools.partial(
    pl.kernel,
    out_type=jax.ShapeDtypeStruct((NC, N_PAD, D), jnp.float32),
    mesh=_mesh,
    scratch_types=[
        pltpu.VMEM((NRB, CH), jnp.int32),
        pltpu.VMEM((NRB + 1, CH), jnp.int32),
        pltpu.VMEM((NRB, CH, D), jnp.float32),
        pltpu.VMEM_SHARED((N_PAD, D), jnp.float32),
        pltpu.SemaphoreType.DMA,
        pltpu.SemaphoreType.DMA,
        pltpu.SemaphoreType.DMA,
    ],
)
def _edge_call(g_hbm, row3_hbm, col3_hbm, out_hbm, ridx_v, cidx_v, rows_v,
               acc_sh, gsem, ssem, isem):
    c = lax.axis_index("c")
    s = lax.axis_index("s")
    wid = c * NS + s
    last = CHUNKS_PER_TILE - 1
    nwch = jnp.where(s < NS - 1, STRIPE // CH, 4)

    pltpu.sync_copy(row3_hbm.at[wid, 0], ridx_v.at[0])
    pltpu.sync_copy(col3_hbm.at[wid, 0], cidx_v.at[0])
    pltpu.async_copy(row3_hbm.at[wid, 1], ridx_v.at[1], isem)
    pltpu.async_copy(col3_hbm.at[wid, 1], cidx_v.at[1], isem)

    def _zero(i, _):
        rows_v[0, i // (D // L), pl.ds((i % (D // L)) * L, L)] = (
            jnp.zeros((L,), jnp.float32))
        return 0
    lax.fori_loop(0, CH * (D // L), _zero, 0)

    def _zchunk(k, _):
        pltpu.sync_copy(rows_v.at[0],
                        acc_sh.at[pl.ds(s * STRIPE + k * CH, CH)])
        return 0
    lax.fori_loop(0, nwch, _zchunk, 0)

    plsc.subcore_barrier()

    pltpu.async_copy(g_hbm.at[ridx_v.at[0]], rows_v.at[0], gsem)

    def _step(j, _):
        buf = jax.lax.rem(j, NRB)
        buf1 = jax.lax.rem(j + 1, NRB)
        rt = jax.lax.rem(j, NRB)
        rt1 = jax.lax.rem(j + 1, NRB)
        rt2 = jax.lax.rem(j + 2, NRB)
        ct = jax.lax.rem(j, NRB + 1)
        ct1 = jax.lax.rem(j + 1, NRB + 1)
        ct2 = jax.lax.rem(j + 2, NRB + 1)

        pltpu.make_async_copy(g_hbm.at[ridx_v.at[rt]], rows_v.at[buf],
                              gsem).wait()

        @pl.when(j > 1)
        def _():
            pltpu.make_async_copy(rows_v.at[buf1],
                                  acc_sh.at[cidx_v.at[ct]], ssem).wait()

        @pl.when(j + 2 <= last)
        def _():
            pltpu.async_copy(row3_hbm.at[wid, j + 2], ridx_v.at[rt2], isem)
            pltpu.async_copy(col3_hbm.at[wid, j + 2], cidx_v.at[ct2], isem)

        @pl.when(j < last)
        def _():
            pltpu.make_async_copy(row3_hbm.at[wid, 0], ridx_v.at[rt1], isem).wait()
            pltpu.make_async_copy(col3_hbm.at[wid, 0], cidx_v.at[ct1], isem).wait()
            pltpu.async_copy(g_hbm.at[ridx_v.at[rt1]], rows_v.at[buf1], gsem)

        pltpu.async_copy(rows_v.at[buf], acc_sh.at[cidx_v.at[ct]], ssem,
                         add=True)
        return 0
    lax.fori_loop(0, CHUNKS_PER_TILE, _step, 0)
    pltpu.make_async_copy(rows_v.at[0], acc_sh.at[cidx_v.at[0]], ssem).wait()
    pltpu.make_async_copy(rows_v.at[0], acc_sh.at[cidx_v.at[0]], ssem).wait()

    plsc.subcore_barrier()

    def _wchunk(k, _):
        r0 = s * STRIPE + k * CH
        pltpu.sync_copy(acc_sh.at[pl.ds(r0, CH)], out_hbm.at[c, pl.ds(r0, CH)])
        return 0
    lax.fori_loop(0, nwch, _wchunk, 0)


def _tc_transform_body(x_ref, w_ref, degp_ref, g_ref):
    deg = degp_ref[0] + degp_ref[1]
    dis = jnp.where(deg > 0, lax.rsqrt(deg), 0.0)
    h = jnp.dot(x_ref[...], w_ref[...], preferred_element_type=jnp.float32)
    hp = jnp.concatenate([h, jnp.zeros((N_PAD - N, D), jnp.float32)], axis=0)
    g_ref[...] = (hp.reshape(N_PAD // D, D, D) * dis[:, :, None]).reshape(N_PAD, D)


def _tc_transform(x, W, degp3):
    return pl.pallas_call(
        _tc_transform_body,
        out_shape=jax.ShapeDtypeStruct((N_PAD, D), jnp.float32),
    )(x, W, degp3)


def _tc_final_body(p_ref, degp_ref, b_ref, o_ref):
    deg = degp_ref[0] + degp_ref[1]
    dis = jnp.where(deg > 0, lax.rsqrt(deg), 0.0)
    tot = p_ref[0] + p_ref[1]
    scaled = (tot.reshape(N_PAD // D, D, D) * dis[:, :, None]).reshape(N_PAD, D)
    o_ref[...] = scaled[:N] + b_ref[...]


def _tc_final(partials, degp3, b2):
    return pl.pallas_call(
        _tc_final_body,
        out_shape=jax.ShapeDtypeStruct((N, D), jnp.float32),
    )(partials, degp3, b2)


def kernel(x, edge_index, W, b):
    ei = edge_index.astype(jnp.int32)
    n_fill = E_PAD - E
    fill = N + (jnp.arange(n_fill, dtype=jnp.int32) % (N_PAD - N))
    row3 = jnp.concatenate([ei[0], fill]).reshape(NC * NS, CHUNKS_PER_TILE, CH)
    col3 = jnp.concatenate([ei[1], fill]).reshape(NC * NS, CHUNKS_PER_TILE, CH)

    degp = _deg_call(col3)
    degp3 = degp.reshape(NC, N_PAD // D, D)
    g = _tc_transform(x, W, degp3)
    partials = _edge_call(g, row3, col3)
    return _tc_final(partials, degp3, b.reshape(1, D))
